# baseline (speedup 1.0000x reference)
import functools

import jax
import jax.numpy as jnp
from jax import lax
from jax.experimental import pallas as pl
from jax.experimental.pallas import tpu as pltpu
from jax.experimental.pallas import tpu_sc as plsc

_NUM_WORKERS = 32
_CHUNK = 1024


def _embed_lookup(table, idx, n, d):
    per_w = n // _NUM_WORKERS
    n_chunks = per_w // _CHUNK
    mesh = plsc.VectorSubcoreMesh(core_axis_name="c", subcore_axis_name="s")

    @functools.partial(
        pl.kernel,
        mesh=mesh,
        out_type=jax.ShapeDtypeStruct((n, d), jnp.float32),
        scratch_types=[
            pltpu.VMEM((_CHUNK,), jnp.int32),
            pltpu.VMEM((_CHUNK, d), jnp.float32),
            pltpu.SemaphoreType.DMA,
        ],
        compiler_params=pltpu.CompilerParams(use_tc_tiling_on_sc=False),
    )
    def k(table_hbm, idx_hbm, out_hbm, idx_v, rows_v, sem):
        wid = lax.axis_index("s") * 2 + lax.axis_index("c")
        base = wid * per_w

        def body(j, carry):
            off = base + j * _CHUNK
            pltpu.sync_copy(idx_hbm.at[pl.ds(off, _CHUNK)], idx_v)
            pltpu.async_copy(table_hbm.at[idx_v], rows_v, sem).wait()
            pltpu.sync_copy(rows_v, out_hbm.at[pl.ds(off, _CHUNK)])
            return carry

        lax.fori_loop(0, n_chunks, body, 0)

    return k(table, idx)


def kernel(seq, table):
    b, h = seq.shape
    _, d = table.shape
    n = b * h
    idx = seq.reshape(n).astype(jnp.int32)
    out = _embed_lookup(table, idx, n, d)
    return out.reshape(b, h, d)


# speedup vs baseline: 1.0250x; 1.0250x over previous
import functools

import jax
import jax.numpy as jnp
from jax import lax
from jax.experimental import pallas as pl
from jax.experimental.pallas import tpu as pltpu
from jax.experimental.pallas import tpu_sc as plsc

_NUM_WORKERS = 32
_CHUNK = 1280
_NBUF = 4


def _embed_lookup(table, idx, n, d):
    per_w = n // _NUM_WORKERS
    n_rounds = per_w // (_CHUNK * _NBUF)
    mesh = plsc.VectorSubcoreMesh(core_axis_name="c", subcore_axis_name="s")

    @functools.partial(
        pl.kernel,
        mesh=mesh,
        out_type=jax.ShapeDtypeStruct((n, d), jnp.float32),
        scratch_types=[
            pltpu.VMEM((per_w,), jnp.int32),
            *[pltpu.VMEM((_CHUNK, d), jnp.float32) for _ in range(_NBUF)],
            *[pltpu.SemaphoreType.DMA for _ in range(_NBUF)],
            *[pltpu.SemaphoreType.DMA for _ in range(_NBUF)],
        ],
        compiler_params=pltpu.CompilerParams(use_tc_tiling_on_sc=False),
    )
    def k(table_hbm, idx_hbm, out_hbm, idx_v, *bufs_and_sems):
        rows = bufs_and_sems[:_NBUF]
        gsem = bufs_and_sems[_NBUF:2 * _NBUF]
        osem = bufs_and_sems[2 * _NBUF:]
        wid = lax.axis_index("s") * 2 + lax.axis_index("c")
        base = wid * per_w

        pltpu.sync_copy(idx_hbm.at[pl.ds(base, per_w)], idx_v)

        def gather(j, b):
            idx_slice = idx_v.at[pl.ds(j * _CHUNK, _CHUNK)]
            pltpu.make_async_copy(table_hbm.at[idx_slice], rows[b],
                                  gsem[b]).start()

        def writeback(j, b):
            dst = out_hbm.at[pl.ds(base + j * _CHUNK, _CHUNK)]
            pltpu.make_async_copy(rows[b], dst, osem[b]).start()

        def wait_gather(j, b):
            idx_slice = idx_v.at[pl.ds(j * _CHUNK, _CHUNK)]
            pltpu.make_async_copy(table_hbm.at[idx_slice], rows[b],
                                  gsem[b]).wait()

        def wait_writeback(j, b):
            dst = out_hbm.at[pl.ds(base + j * _CHUNK, _CHUNK)]
            pltpu.make_async_copy(rows[b], dst, osem[b]).wait()

        def round_body(g, carry):
            for b in range(_NBUF):
                j = g * _NBUF + b

                @pl.when(g > 0)
                def _(b=b, j=j):
                    wait_writeback(j, b)

                gather(j, b)
            for b in range(_NBUF):
                j = g * _NBUF + b
                wait_gather(j, b)
                writeback(j, b)
            return carry

        lax.fori_loop(0, n_rounds, round_body, 0)
        last = (n_rounds - 1) * _NBUF
        for b in range(_NBUF):
            wait_writeback(last + b, b)

    return k(table, idx)


def kernel(seq, table):
    b, h = seq.shape
    _, d = table.shape
    n = b * h
    idx = seq.reshape(n).astype(jnp.int32)
    out = _embed_lookup(table, idx, n, d)
    return out.reshape(b, h, d)
